# Initial kernel scaffold; baseline (speedup 1.0000x reference)
import jax
import jax.numpy as jnp
from jax.experimental import pallas as pl


def kernel(hidden_states, token_ids, gate_proj_w, up_proj_w, down_proj_w, shared_gate_w, shared_up_w, shared_down_w):
    raise NotImplementedError("write your pallas kernel here")


# speedup vs baseline: 2.6336x; 2.6336x over previous
import functools

import jax
import jax.numpy as jnp
from jax import lax
from jax.experimental import pallas as pl
from jax.experimental.pallas import tpu as pltpu
from jax.experimental.pallas import tpu_sc as plsc

_T = 256
_IB = 1024


def _sc_gather_rows(table, idx):
    rows, hdim = table.shape
    (k,) = idx.shape
    info = plsc.get_sparse_core_info()
    nw = info.num_cores * info.num_subcores
    rpw = k // nw
    ch = 32
    nch = rpw // ch
    assert rpw % ch == 0 and k % nw == 0

    mesh = plsc.VectorSubcoreMesh(core_axis_name="c", subcore_axis_name="s")

    @functools.partial(
        pl.kernel,
        mesh=mesh,
        out_type=jax.ShapeDtypeStruct((k, hdim), table.dtype),
        scratch_types=[
            pltpu.VMEM((ch,), jnp.int32),
            pltpu.VMEM((ch, hdim), table.dtype),
            pltpu.SemaphoreType.DMA,
        ],
    )
    def gather_kernel(table_hbm, idx_hbm, out_hbm, idx_v, rows_v, sem):
        wid = lax.axis_index("s") * info.num_cores + lax.axis_index("c")
        base0 = wid * rpw
        for t in range(nch):
            base = base0 + t * ch
            pltpu.sync_copy(idx_hbm.at[pl.ds(base, ch)], idx_v)
            pltpu.async_copy(table_hbm.at[idx_v], rows_v, sem).wait()
            pltpu.sync_copy(rows_v, out_hbm.at[pl.ds(base, ch)])

    return gather_kernel(table, idx)


def _routed_interm(x_pad, gate_w, up_w, block_expert):
    p, h = x_pad.shape
    e, _, i_e = gate_w.shape
    npb = p // _T
    ni = i_e // _IB

    def body(be_ref, x_ref, g_ref, u_ref, o_ref):
        x = x_ref[...]
        g = jnp.dot(x, g_ref[0], preferred_element_type=jnp.float32)
        u = jnp.dot(x, u_ref[0], preferred_element_type=jnp.float32)
        o_ref[...] = g * jax.nn.sigmoid(g) * u

    grid_spec = pltpu.PrefetchScalarGridSpec(
        num_scalar_prefetch=1,
        grid=(ni, npb),
        in_specs=[
            pl.BlockSpec((_T, h), lambda i, b, be: (b, 0)),
            pl.BlockSpec((1, h, _IB), lambda i, b, be: (be[b], 0, i)),
            pl.BlockSpec((1, h, _IB), lambda i, b, be: (be[b], 0, i)),
        ],
        out_specs=pl.BlockSpec((_T, _IB), lambda i, b, be: (b, i)),
    )
    return pl.pallas_call(
        body,
        grid_spec=grid_spec,
        out_shape=jax.ShapeDtypeStruct((p, i_e), jnp.float32),
        compiler_params=pltpu.CompilerParams(
            dimension_semantics=("arbitrary", "arbitrary")
        ),
    )(block_expert, x_pad, gate_w, up_w)


def _routed_down(inter_pad, down_w, block_expert):
    p, i_e = inter_pad.shape
    e, _, h = down_w.shape
    npb = p // _T

    def body(be_ref, a_ref, d_ref, o_ref):
        o_ref[...] = jnp.dot(a_ref[...], d_ref[0], preferred_element_type=jnp.float32)

    grid_spec = pltpu.PrefetchScalarGridSpec(
        num_scalar_prefetch=1,
        grid=(npb,),
        in_specs=[
            pl.BlockSpec((_T, i_e), lambda b, be: (b, 0)),
            pl.BlockSpec((1, i_e, h), lambda b, be: (be[b], 0, 0)),
        ],
        out_specs=pl.BlockSpec((_T, h), lambda b, be: (b, 0)),
    )
    return pl.pallas_call(
        body,
        grid_spec=grid_spec,
        out_shape=jax.ShapeDtypeStruct((p, h), jnp.float32),
        compiler_params=pltpu.CompilerParams(dimension_semantics=("arbitrary",)),
    )(block_expert, inter_pad, down_w)


def _shared_interm(flat, gate_w, up_w):
    n, h = flat.shape
    _, i_e = gate_w.shape
    nb = n // _T
    ni = i_e // _IB

    def body(x_ref, g_ref, u_ref, o_ref):
        x = x_ref[...]
        g = jnp.dot(x, g_ref[...], preferred_element_type=jnp.float32)
        u = jnp.dot(x, u_ref[...], preferred_element_type=jnp.float32)
        o_ref[...] = g * jax.nn.sigmoid(g) * u

    return pl.pallas_call(
        body,
        grid=(ni, nb),
        in_specs=[
            pl.BlockSpec((_T, h), lambda i, b: (b, 0)),
            pl.BlockSpec((h, _IB), lambda i, b: (0, i)),
            pl.BlockSpec((h, _IB), lambda i, b: (0, i)),
        ],
        out_specs=pl.BlockSpec((_T, _IB), lambda i, b: (b, i)),
        out_shape=jax.ShapeDtypeStruct((n, i_e), jnp.float32),
        compiler_params=pltpu.CompilerParams(
            dimension_semantics=("arbitrary", "arbitrary")
        ),
    )(flat, gate_w, up_w)


def _shared_down_add(inter_s, down_w, routed):
    n, i_e = inter_s.shape
    _, h = down_w.shape
    nb = n // _T

    def body(a_ref, d_ref, r_ref, o_ref):
        o_ref[...] = r_ref[...] + jnp.dot(
            a_ref[...], d_ref[...], preferred_element_type=jnp.float32
        )

    return pl.pallas_call(
        body,
        grid=(nb,),
        in_specs=[
            pl.BlockSpec((_T, i_e), lambda b: (b, 0)),
            pl.BlockSpec((i_e, h), lambda b: (0, 0)),
            pl.BlockSpec((_T, h), lambda b: (b, 0)),
        ],
        out_specs=pl.BlockSpec((_T, h), lambda b: (b, 0)),
        out_shape=jax.ShapeDtypeStruct((n, h), jnp.float32),
        compiler_params=pltpu.CompilerParams(dimension_semantics=("arbitrary",)),
    )(inter_s, down_w, routed)


def _routing_metadata(token_ids, num_experts, vocab, n, p):
    tid = jnp.clip(token_ids, 0, vocab - 1).reshape(-1)
    eid = (tid % num_experts).astype(jnp.int32)
    onehot = (eid[:, None] == jnp.arange(num_experts, dtype=jnp.int32)[None, :])
    counts = jnp.sum(onehot.astype(jnp.int32), axis=0)
    rank = jnp.cumsum(onehot.astype(jnp.int32), axis=0) - 1
    rank_i = jnp.take_along_axis(rank, eid[:, None], axis=1)[:, 0]
    padded = ((counts + _T - 1) // _T) * _T
    pad_off = jnp.concatenate(
        [jnp.zeros((1,), jnp.int32), jnp.cumsum(padded).astype(jnp.int32)]
    )
    dest = pad_off[eid] + rank_i
    src = jnp.zeros((p,), jnp.int32).at[dest].set(
        jnp.arange(n, dtype=jnp.int32)
    )
    npb = p // _T
    block_expert = jnp.clip(
        jnp.searchsorted(
            pad_off[1:], jnp.arange(npb, dtype=jnp.int32) * _T, side="right"
        ),
        0,
        num_experts - 1,
    ).astype(jnp.int32)
    return src, dest, block_expert


def kernel(hidden_states, token_ids, gate_proj_w, up_proj_w, down_proj_w,
           shared_gate_w, shared_up_w, shared_down_w):
    b, s, h = hidden_states.shape
    e = gate_proj_w.shape[0]
    vocab = 100000
    n = b * s
    p = n + e * _T

    flat = hidden_states.reshape(n, h)
    src, dest, block_expert = _routing_metadata(token_ids, e, vocab, n, p)

    x_pad = _sc_gather_rows(flat, src)
    inter_pad = _routed_interm(x_pad, gate_proj_w, up_proj_w, block_expert)
    y_pad = _routed_down(inter_pad, down_proj_w, block_expert)
    routed = _sc_gather_rows(y_pad, dest)

    inter_s = _shared_interm(flat, shared_gate_w, shared_up_w)
    out = _shared_down_add(inter_s, shared_down_w, routed)
    return out.reshape(b, s, h)
